# initial kernel scaffold (unmeasured)
import jax
import jax.numpy as jnp
from jax import lax
from jax.experimental import pallas as pl
from jax.experimental.pallas import tpu as pltpu

N_DEV = 4


def kernel(x, w_mat):
    m_total, k_per = x.shape
    k_total, n = w_mat.shape
    m_per = m_total // N_DEV

    def body(x_ref, w_ref, out_ref, recv_buf, amax_send, amax_recv,
             x_send_sems, x_recv_sems, a_send_sems, a_recv_sems):
        my = lax.axis_index("i")

        barrier_sem = pltpu.get_barrier_semaphore()
        for d in range(1, N_DEV):
            peer = (my + d) % N_DEV
            pl.semaphore_signal(
                barrier_sem, inc=1,
                device_id=(peer,), device_id_type=pl.DeviceIdType.MESH,
            )
        pl.semaphore_wait(barrier_sem, N_DEV - 1)

        rdmas = []
        for d in range(1, N_DEV):
            tgt = (my + d) % N_DEV
            rdma = pltpu.make_async_remote_copy(
                src_ref=x_ref.at[pl.ds(tgt * m_per, m_per), :],
                dst_ref=recv_buf.at[d - 1],
                send_sem=x_send_sems.at[d - 1],
                recv_sem=x_recv_sems.at[d - 1],
                device_id=(tgt,),
                device_id_type=pl.DeviceIdType.MESH,
            )
            rdma.start()
            rdmas.append(rdma)

        out_ref[:, :] = jnp.dot(
            x_ref[pl.ds(my * m_per, m_per), :],
            w_ref[pl.ds(my * k_per, k_per), :],
            preferred_element_type=jnp.float32,
        )

        for d in (1, 3, 2):
            rdmas[d - 1].wait_recv()
            src = (my + (N_DEV - d)) % N_DEV
            out_ref[:, :] += jnp.dot(
                recv_buf[d - 1],
                w_ref[pl.ds(src * k_per, k_per), :],
                preferred_element_type=jnp.float32,
            )
        for d in range(1, N_DEV):
            rdmas[d - 1].wait_send()

        lam = jnp.max(jnp.abs(out_ref[:, :]))
        amax_send[0, :] = jnp.full((128,), lam, jnp.float32)
        a_rdmas = []
        for d in range(1, N_DEV):
            tgt = (my + d) % N_DEV
            ar = pltpu.make_async_remote_copy(
                src_ref=amax_send,
                dst_ref=amax_recv.at[d - 1],
                send_sem=a_send_sems.at[d - 1],
                recv_sem=a_recv_sems.at[d - 1],
                device_id=(tgt,),
                device_id_type=pl.DeviceIdType.MESH,
            )
            ar.start()
            a_rdmas.append(ar)
        gmax = lam
        for d in range(1, N_DEV):
            a_rdmas[d - 1].wait()
            gmax = jnp.maximum(gmax, amax_recv[d - 1, 0, 0])

        scale = gmax / 127.0
        q = jnp.clip(jnp.round(out_ref[:, :] / scale), -127.0, 127.0)
        out_ref[:, :] = q * scale

    return pl.pallas_call(
        body,
        out_shape=jax.ShapeDtypeStruct((m_per, n), jnp.float32),
        in_specs=[
            pl.BlockSpec(memory_space=pltpu.VMEM),
            pl.BlockSpec(memory_space=pltpu.VMEM),
        ],
        out_specs=pl.BlockSpec(memory_space=pltpu.VMEM),
        scratch_shapes=[
            pltpu.VMEM((N_DEV - 1, m_per, k_per), x.dtype),
            pltpu.VMEM((1, 128), jnp.float32),
            pltpu.VMEM((N_DEV - 1, 1, 128), jnp.float32),
            pltpu.SemaphoreType.DMA((N_DEV - 1,)),
            pltpu.SemaphoreType.DMA((N_DEV - 1,)),
            pltpu.SemaphoreType.DMA((N_DEV - 1,)),
            pltpu.SemaphoreType.DMA((N_DEV - 1,)),
        ],
        compiler_params=pltpu.CompilerParams(collective_id=0),
    )(x, w_mat)


# baseline (device time: 97724 ns/iter reference)
import jax
import jax.numpy as jnp
from jax import lax
from jax.experimental import pallas as pl
from jax.experimental.pallas import tpu as pltpu

N_DEV = 4


def kernel(x, w_mat):
    m_total, k_per = x.shape
    k_total, n = w_mat.shape
    m_per = m_total // N_DEV
    x = x.astype(jnp.bfloat16)
    w_mat = w_mat.astype(jnp.bfloat16)

    def body(x_ref, w_ref, out_ref, recv_buf, amax_send, amax_recv,
             x_send_sems, x_recv_sems, a_send_sems, a_recv_sems):
        my = lax.axis_index("i")

        barrier_sem = pltpu.get_barrier_semaphore()
        for d in range(1, N_DEV):
            peer = (my + d) % N_DEV
            pl.semaphore_signal(
                barrier_sem, inc=1,
                device_id=(peer,), device_id_type=pl.DeviceIdType.MESH,
            )
        pl.semaphore_wait(barrier_sem, N_DEV - 1)

        rdmas = []
        for d in range(1, N_DEV):
            tgt = (my + d) % N_DEV
            rdma = pltpu.make_async_remote_copy(
                src_ref=x_ref.at[pl.ds(tgt * m_per, m_per), :],
                dst_ref=recv_buf.at[d - 1],
                send_sem=x_send_sems.at[d - 1],
                recv_sem=x_recv_sems.at[d - 1],
                device_id=(tgt,),
                device_id_type=pl.DeviceIdType.MESH,
            )
            rdma.start()
            rdmas.append(rdma)

        out_ref[:, :] = jnp.dot(
            x_ref[pl.ds(my * m_per, m_per), :],
            w_ref[pl.ds(my * k_per, k_per), :],
            preferred_element_type=jnp.float32,
        )

        for d in (1, 3, 2):
            rdmas[d - 1].wait_recv()
            src = (my + (N_DEV - d)) % N_DEV
            out_ref[:, :] += jnp.dot(
                recv_buf[d - 1],
                w_ref[pl.ds(src * k_per, k_per), :],
                preferred_element_type=jnp.float32,
            )
        for d in range(1, N_DEV):
            rdmas[d - 1].wait_send()

        lam = jnp.max(jnp.abs(out_ref[:, :]))
        amax_send[0, :] = jnp.full((128,), lam, jnp.float32)
        a_rdmas = []
        for d in range(1, N_DEV):
            tgt = (my + d) % N_DEV
            ar = pltpu.make_async_remote_copy(
                src_ref=amax_send,
                dst_ref=amax_recv.at[d - 1],
                send_sem=a_send_sems.at[d - 1],
                recv_sem=a_recv_sems.at[d - 1],
                device_id=(tgt,),
                device_id_type=pl.DeviceIdType.MESH,
            )
            ar.start()
            a_rdmas.append(ar)
        gmax = lam
        for d in range(1, N_DEV):
            a_rdmas[d - 1].wait()
            gmax = jnp.maximum(gmax, amax_recv[d - 1, 0, 0])

        scale = gmax / 127.0
        q = jnp.clip(jnp.round(out_ref[:, :] / scale), -127.0, 127.0)
        out_ref[:, :] = q * scale

    return pl.pallas_call(
        body,
        out_shape=jax.ShapeDtypeStruct((m_per, n), jnp.float32),
        in_specs=[
            pl.BlockSpec(memory_space=pltpu.VMEM),
            pl.BlockSpec(memory_space=pltpu.VMEM),
        ],
        out_specs=pl.BlockSpec(memory_space=pltpu.VMEM),
        scratch_shapes=[
            pltpu.VMEM((N_DEV - 1, m_per, k_per), x.dtype),
            pltpu.VMEM((1, 128), jnp.float32),
            pltpu.VMEM((N_DEV - 1, 1, 128), jnp.float32),
            pltpu.SemaphoreType.DMA((N_DEV - 1,)),
            pltpu.SemaphoreType.DMA((N_DEV - 1,)),
            pltpu.SemaphoreType.DMA((N_DEV - 1,)),
            pltpu.SemaphoreType.DMA((N_DEV - 1,)),
        ],
        compiler_params=pltpu.CompilerParams(collective_id=0),
    )(x, w_mat)


# device time: 78352 ns/iter; 1.2472x vs baseline; 1.2472x over previous
import jax
import jax.numpy as jnp
from jax import lax
from jax.experimental import pallas as pl
from jax.experimental.pallas import tpu as pltpu

N_DEV = 4


def kernel(x, w_mat):
    m_total, k_per = x.shape
    k_total, n = w_mat.shape
    m_per = m_total // N_DEV

    def body(x_hbm, w_hbm, out_ref, x16, w16, recv_buf, stage,
             amax_send, amax_recv,
             x_send_sems, x_recv_sems, a_send_sems, a_recv_sems, copy_sems):
        my = lax.axis_index("i")

        send_ds = (1, 3, 2)
        x_blocks = [(my + d) % N_DEV for d in send_ds] + [my]
        w_chunks = [my, (my + 3) % N_DEV, (my + 1) % N_DEV, (my + 2) % N_DEV]

        def make_copy(i, slot):
            if i < 4:
                blk = x_blocks[i]
                return pltpu.make_async_copy(
                    x_hbm.at[pl.ds(blk * m_per, m_per), :],
                    stage.at[slot, :, pl.ds(0, k_per)],
                    copy_sems.at[slot],
                )
            ch = w_chunks[i - 4]
            return pltpu.make_async_copy(
                w_hbm.at[pl.ds(ch * k_per, k_per), :],
                stage.at[slot],
                copy_sems.at[slot],
            )

        barrier_sem = pltpu.get_barrier_semaphore()
        for d in send_ds:
            pl.semaphore_signal(
                barrier_sem, inc=1,
                device_id=((my + d) % N_DEV,),
                device_id_type=pl.DeviceIdType.MESH,
            )
        cps = {}
        for i in (0, 1):
            cps[i] = make_copy(i, i)
            cps[i].start()
        pl.semaphore_wait(barrier_sem, N_DEV - 1)

        rdmas = {}
        for i in range(8):
            slot = i % 2
            cps[i].wait()
            chunk32 = stage[slot, :, pl.ds(0, k_per)] if i < 4 else stage[slot]
            if i < 4:
                x16[pl.ds(x_blocks[i] * m_per, m_per), :] = (
                    chunk32.astype(jnp.bfloat16))
            else:
                w16[pl.ds(w_chunks[i - 4] * k_per, k_per), :] = (
                    chunk32.astype(jnp.bfloat16))
            if i + 2 < 8:
                cps[i + 2] = make_copy(i + 2, slot)
                cps[i + 2].start()
            if i < 3:
                d = send_ds[i]
                rdma = pltpu.make_async_remote_copy(
                    src_ref=x16.at[pl.ds(x_blocks[i] * m_per, m_per), :],
                    dst_ref=recv_buf.at[d - 1],
                    send_sem=x_send_sems.at[d - 1],
                    recv_sem=x_recv_sems.at[d - 1],
                    device_id=(x_blocks[i],),
                    device_id_type=pl.DeviceIdType.MESH,
                )
                rdma.start()
                rdmas[d] = rdma
            elif i == 4:
                out_ref[:, :] = jnp.dot(
                    x16[pl.ds(my * m_per, m_per), :],
                    w16[pl.ds(my * k_per, k_per), :],
                    preferred_element_type=jnp.float32,
                )
            elif i >= 5:
                d = send_ds[i - 5]
                rdmas[d].wait_recv()
                out_ref[:, :] += jnp.dot(
                    recv_buf[d - 1],
                    w16[pl.ds(w_chunks[i - 4] * k_per, k_per), :],
                    preferred_element_type=jnp.float32,
                )
        for d in send_ds:
            rdmas[d].wait_send()

        lam = jnp.max(jnp.abs(out_ref[:, :]))
        amax_send[0, :] = jnp.full((128,), lam, jnp.float32)
        a_rdmas = []
        for d in send_ds:
            ar = pltpu.make_async_remote_copy(
                src_ref=amax_send,
                dst_ref=amax_recv.at[d - 1],
                send_sem=a_send_sems.at[d - 1],
                recv_sem=a_recv_sems.at[d - 1],
                device_id=((my + d) % N_DEV,),
                device_id_type=pl.DeviceIdType.MESH,
            )
            ar.start()
            a_rdmas.append(ar)
        gmax = lam
        for i, d in enumerate(send_ds):
            a_rdmas[i].wait()
            gmax = jnp.maximum(gmax, amax_recv[d - 1, 0, 0])

        scale = gmax / 127.0
        q = jnp.clip(jnp.round(out_ref[:, :] / scale), -127.0, 127.0)
        out_ref[:, :] = q * scale

    return pl.pallas_call(
        body,
        out_shape=jax.ShapeDtypeStruct((m_per, n), jnp.float32),
        in_specs=[
            pl.BlockSpec(memory_space=pl.ANY),
            pl.BlockSpec(memory_space=pl.ANY),
        ],
        out_specs=pl.BlockSpec(memory_space=pltpu.VMEM),
        scratch_shapes=[
            pltpu.VMEM((m_total, k_per), jnp.bfloat16),
            pltpu.VMEM((k_total, n), jnp.bfloat16),
            pltpu.VMEM((N_DEV - 1, m_per, k_per), jnp.bfloat16),
            pltpu.VMEM((2, m_per, n), jnp.float32),
            pltpu.VMEM((1, 128), jnp.float32),
            pltpu.VMEM((N_DEV - 1, 1, 128), jnp.float32),
            pltpu.SemaphoreType.DMA((N_DEV - 1,)),
            pltpu.SemaphoreType.DMA((N_DEV - 1,)),
            pltpu.SemaphoreType.DMA((N_DEV - 1,)),
            pltpu.SemaphoreType.DMA((N_DEV - 1,)),
            pltpu.SemaphoreType.DMA((2,)),
        ],
        compiler_params=pltpu.CompilerParams(
            collective_id=0,
            vmem_limit_bytes=63 * 1024 * 1024,
        ),
    )(x, w_mat)


# device time: 72629 ns/iter; 1.3455x vs baseline; 1.0788x over previous
import jax
import jax.numpy as jnp
from jax import lax
from jax.experimental import pallas as pl
from jax.experimental.pallas import tpu as pltpu

N_DEV = 4


def kernel(x, w_mat):
    m_total, k_per = x.shape
    k_total, n = w_mat.shape
    m_per = m_total // N_DEV
    hk = k_per // 2
    qk = k_per // 4
    rm = m_per // 2

    def body(x_hbm, w_hbm, out_ref, x16, w16, recv_buf, stage,
             amax_send, amax_recv,
             x_send_sems, x_recv_sems, a_send_sems, a_recv_sems, copy_sems):
        my = lax.axis_index("i")
        right = (my + 1) % N_DEV
        left = (my + 3) % N_DEV
        diag = (my + 2) % N_DEV

        x_tasks = [(right, 0), (left, 0), (right, 1), (left, 1),
                   (diag, 0), (diag, 1), (my, 0), (my, 1)]
        w_chunks = [my, left, right, diag]
        n_tasks = len(x_tasks) + len(w_chunks)

        def make_copy(i, slot):
            if i < len(x_tasks):
                blk, h = x_tasks[i]
                return pltpu.make_async_copy(
                    x_hbm.at[pl.ds(blk * m_per, m_per), pl.ds(h * hk, hk)],
                    stage.at[slot, :, pl.ds(0, hk)],
                    copy_sems.at[slot],
                )
            ch = w_chunks[i - len(x_tasks)]
            return pltpu.make_async_copy(
                w_hbm.at[pl.ds(ch * k_per, k_per), :],
                stage.at[slot],
                copy_sems.at[slot],
            )

        barrier_sem = pltpu.get_barrier_semaphore()
        for peer in (right, left, diag):
            pl.semaphore_signal(
                barrier_sem, inc=1,
                device_id=(peer,), device_id_type=pl.DeviceIdType.MESH,
            )
        cps = {}
        for i in (0, 1):
            cps[i] = make_copy(i, i)
            cps[i].start()
        pl.semaphore_wait(barrier_sem, N_DEV - 1)

        rdmas = {}

        a_rdmas = {}

        def a_round(r):
            for pi, peer in enumerate((right, left, diag)):
                sem = 2 * pi + r
                ar = pltpu.make_async_remote_copy(
                    src_ref=amax_send.at[pl.ds(r, 1)],
                    dst_ref=amax_recv.at[pi, pl.ds(r, 1)],
                    send_sem=a_send_sems.at[sem],
                    recv_sem=a_recv_sems.at[sem],
                    device_id=(peer,),
                    device_id_type=pl.DeviceIdType.MESH,
                )
                ar.start()
                a_rdmas[sem] = ar

        def send(src_col, width, dst_slot, dst_col, sem, tgt):
            r = pltpu.make_async_remote_copy(
                src_ref=x16.at[pl.ds(tgt * m_per, m_per),
                               pl.ds(src_col, width)],
                dst_ref=recv_buf.at[dst_slot, :, pl.ds(dst_col, width)],
                send_sem=x_send_sems.at[sem],
                recv_sem=x_recv_sems.at[sem],
                device_id=(tgt,),
                device_id_type=pl.DeviceIdType.MESH,
            )
            r.start()
            rdmas[sem] = r

        for i in range(n_tasks):
            slot = i % 2
            cps[i].wait()
            if i < len(x_tasks):
                blk, h = x_tasks[i]
                x16[pl.ds(blk * m_per, m_per), pl.ds(h * hk, hk)] = (
                    stage[slot, :, pl.ds(0, hk)].astype(jnp.bfloat16))
            else:
                ch = w_chunks[i - len(x_tasks)]
                w16[pl.ds(ch * k_per, k_per), :] = (
                    stage[slot].astype(jnp.bfloat16))
            if i + 2 < n_tasks:
                cps[i + 2] = make_copy(i + 2, slot)
                cps[i + 2].start()

            if i < 4:
                blk, h = x_tasks[i]
                to_right = (i % 2 == 0)
                send(h * hk, hk,
                     0 if to_right else 1, h * hk,
                     h if to_right else 6 + h,
                     blk)
            elif i == 5:
                for q in range(4):
                    rq = m_per // 4
                    r = pltpu.make_async_remote_copy(
                        src_ref=x16.at[pl.ds(diag * m_per + q * rq, rq), :],
                        dst_ref=recv_buf.at[2, pl.ds(q * rq, rq), :],
                        send_sem=x_send_sems.at[2 + q],
                        recv_sem=x_recv_sems.at[2 + q],
                        device_id=(diag,),
                        device_id_type=pl.DeviceIdType.MESH,
                    )
                    r.start()
                    rdmas[2 + q] = r
            elif i == 8:
                out_ref[:, :] = jnp.dot(
                    x16[pl.ds(my * m_per, m_per), :],
                    w16[pl.ds(my * k_per, k_per), :],
                    preferred_element_type=jnp.float32,
                )
            elif i in (9, 10):
                src_slot = i - 9
                kblk = w_chunks[i - 8]
                for h in (0, 1):
                    rdmas[(0 if src_slot == 0 else 6) + h].wait_recv()
                out_ref[:, :] += jnp.dot(
                    recv_buf[src_slot],
                    w16[pl.ds(kblk * k_per, k_per), :],
                    preferred_element_type=jnp.float32,
                )
            elif i == 11:
                wrow = w_chunks[3] * k_per
                rq = m_per // 4
                pmax = []
                for q in range(4):
                    rdmas[2 + q].wait_recv()
                    out_ref[pl.ds(q * rq, rq), :] += jnp.dot(
                        recv_buf[2, pl.ds(q * rq, rq), :],
                        w16[pl.ds(wrow, k_per), :],
                        preferred_element_type=jnp.float32,
                    )
                    pmax.append(
                        jnp.max(jnp.abs(out_ref[pl.ds(q * rq, rq), :])))
                lam0 = jnp.maximum(pmax[0], pmax[1])
                lam1 = jnp.maximum(pmax[2], pmax[3])

        for r in rdmas.values():
            r.wait_send()
        gmax = jnp.maximum(lam0, lam1)
        amax_send[0, :] = jnp.full((128,), gmax, jnp.float32)
        a_round(0)
        for sem, ar in a_rdmas.items():
            ar.wait()
            gmax = jnp.maximum(gmax, amax_recv[sem // 2, sem % 2, 0])

        scale = gmax / 127.0
        q = jnp.clip(jnp.round(out_ref[:, :] / scale), -127.0, 127.0)
        out_ref[:, :] = q * scale

    return pl.pallas_call(
        body,
        out_shape=jax.ShapeDtypeStruct((m_per, n), jnp.float32),
        in_specs=[
            pl.BlockSpec(memory_space=pl.ANY),
            pl.BlockSpec(memory_space=pl.ANY),
        ],
        out_specs=pl.BlockSpec(memory_space=pltpu.VMEM),
        scratch_shapes=[
            pltpu.VMEM((m_total, k_per), jnp.bfloat16),
            pltpu.VMEM((k_total, n), jnp.bfloat16),
            pltpu.VMEM((3, m_per, k_per), jnp.bfloat16),
            pltpu.VMEM((2, m_per, n), jnp.float32),
            pltpu.VMEM((2, 128), jnp.float32),
            pltpu.VMEM((N_DEV - 1, 2, 128), jnp.float32),
            pltpu.SemaphoreType.DMA((8,)),
            pltpu.SemaphoreType.DMA((8,)),
            pltpu.SemaphoreType.DMA((6,)),
            pltpu.SemaphoreType.DMA((6,)),
            pltpu.SemaphoreType.DMA((2,)),
        ],
        compiler_params=pltpu.CompilerParams(
            collective_id=0,
            vmem_limit_bytes=63 * 1024 * 1024,
        ),
    )(x, w_mat)
